# baseline (speedup 1.0000x reference)
import functools

import jax
import jax.numpy as jnp
from jax import lax
from jax.experimental import pallas as pl
from jax.experimental.pallas import tpu as pltpu
from jax.experimental.pallas import tpu_sc as plsc

B = 16384
L = 50
H = 64
V = 100000
PAD_IDX = 0
EPS = 1e-8

NC = 2
NS = 16
NW = NC * NS
LN = 16
HK = H // LN
IL = 4

RB = 8
CH = RB * L
GSZ = 80
NG = CH // GSZ
BPW = B // NW
NCHUNK = BPW // RB
TOKH = NCHUNK // 2


def _rsqrt_nr_multi(xs):
    ii = [lax.bitcast_convert_type(x, jnp.int32) for x in xs]
    ii = [jnp.int32(0x5F3759DF) - lax.shift_right_logical(i, 1) for i in ii]
    ys = [lax.bitcast_convert_type(i, jnp.float32) for i in ii]
    halves = [0.5 * x for x in xs]
    ys = [y * (1.5 - h * y * y) for y, h in zip(ys, halves)]
    return ys


def _sc_body(tok_hbm, word_hbm, posgb_hbm, out_hbm,
             tok_v, in_a, in_b, out_a, out_b, pos_v,
             gsem_a, gsem_b, osem_a, osem_b):
    wid = lax.axis_index("s") * NC + lax.axis_index("c")
    b0 = wid * BPW

    pltpu.sync_copy(tok_hbm.at[wid, 0], tok_v)
    pltpu.sync_copy(posgb_hbm, pos_v)

    gdnums = lax.GatherDimensionNumbers(
        offset_dims=(), collapsed_slice_dims=(0,), start_index_map=(0,))

    def lane_shuffle(x, idx):
        return lax.gather(
            x, idx.reshape(LN, 1), gdnums, (1,), unique_indices=True,
            indices_are_sorted=False,
            mode=lax.GatherScatterMode.PROMISE_IN_BOUNDS)

    gvecs = [pos_v[L, pl.ds(k * LN, LN)] for k in range(HK)]
    bvecs = [pos_v[L + 1, pl.ds(k * LN, LN)] for k in range(HK)]
    last_lane = jnp.full((LN,), LN - 1, jnp.int32)

    def start_gathers(ci, in_ref, gsem):
        cm = lax.rem(ci, TOKH)
        for g5 in range(NG):
            pltpu.async_copy(word_hbm.at[tok_v.at[cm, pl.ds(g5 * GSZ, GSZ)]],
                             in_ref.at[pl.ds(g5 * GSZ, GSZ)], gsem)

    def wait_gathers(ci, in_ref, gsem):
        cm = lax.rem(ci, TOKH)
        for g5 in range(NG):
            pltpu.make_async_copy(
                word_hbm.at[tok_v.at[cm, pl.ds(g5 * GSZ, GSZ)]],
                in_ref.at[pl.ds(g5 * GSZ, GSZ)], gsem).wait()

    def start_put(ci, out_ref, osem):
        return pltpu.async_copy(
            out_ref, out_hbm.at[pl.ds(b0 + ci * RB, RB)], osem)

    def wait_put(ci, out_ref, osem):
        pltpu.make_async_copy(
            out_ref, out_hbm.at[pl.ds(b0 + ci * RB, RB)], osem).wait()

    def compute_chunk(ci, in_ref, out_ref):
        cm = lax.rem(ci, TOKH)

        def ln_rows(gi, tokg, masked):
            maskf_g = (jnp.where(tokg != PAD_IDX, 1.0, 0.0) if masked
                       else None)
            gbase = gi * LN
            for blk in range(LN // IL):
                rs = [blk * IL + t for t in range(IL)]
                iidx = [gbase + r for r in rs]
                rbs = [lax.div(i, L) for i in iidx]
                lps = [lax.rem(i, L) for i in iidx]
                E = [[in_ref[i, pl.ds(k * LN, LN)]
                      + pos_v[lp, pl.ds(k * LN, LN)] for k in range(HK)]
                     for i, lp in zip(iidx, lps)]
                S = [(e[0] + e[1]) + (e[2] + e[3]) for e in E]
                Q = [(e[0] * e[0] + e[1] * e[1]) + (e[2] * e[2] + e[3] * e[3])
                     for e in E]
                S = [plsc.cumsum(s) for s in S]
                Q = [plsc.cumsum(q) for q in Q]
                S = [lane_shuffle(s, last_lane) for s in S]
                Q = [lane_shuffle(q, last_lane) for q in Q]
                means = [s * (1.0 / H) for s in S]
                vars_ = [q * (1.0 / H) - m * m for q, m in zip(Q, means)]
                rstds = _rsqrt_nr_multi([v + EPS for v in vars_])
                if masked:
                    mfs = [lane_shuffle(maskf_g, jnp.full((LN,), r, jnp.int32))
                           for r in rs]
                    As = [rv * mf for rv, mf in zip(rstds, mfs)]
                else:
                    As = rstds
                Cs = [m * a for m, a in zip(means, As)]
                for t in range(IL):
                    for k in range(HK):
                        o = (E[t][k] * As[t] - Cs[t]) * gvecs[k]
                        o = o + bvecs[k] * mfs[t] if masked else o + bvecs[k]
                        out_ref[rbs[t], lps[t], pl.ds(k * LN, LN)] = o

        def group_body(gi, _):
            tokg = tok_v[cm, pl.ds(gi * LN, LN)]
            npad = plsc.all_reduce_population_count(tokg == PAD_IDX)[0]

            @pl.when(npad == 0)
            def _():
                ln_rows(gi, tokg, masked=False)

            @pl.when(npad != 0)
            def _():
                ln_rows(gi, tokg, masked=True)
            return 0

        lax.fori_loop(0, CH // LN, group_body, 0)

    start_gathers(0, in_a, gsem_a)
    start_gathers(1, in_b, gsem_b)

    def pair_body(c2, _):
        ci_a = c2 * 2
        ci_b = ci_a + 1

        @pl.when(c2 == TOKH // 2 - 1)
        def _():
            pltpu.sync_copy(tok_hbm.at[wid, 1], tok_v)

        wait_gathers(ci_a, in_a, gsem_a)

        @pl.when(c2 > 0)
        def _():
            wait_put(ci_a - 2, out_a, osem_a)

        compute_chunk(ci_a, in_a, out_a)

        @pl.when(ci_a + 2 < NCHUNK)
        def _():
            start_gathers(ci_a + 2, in_a, gsem_a)
        start_put(ci_a, out_a, osem_a)

        wait_gathers(ci_b, in_b, gsem_b)

        @pl.when(c2 > 0)
        def _():
            wait_put(ci_b - 2, out_b, osem_b)

        compute_chunk(ci_b, in_b, out_b)

        @pl.when(ci_b + 2 < NCHUNK)
        def _():
            start_gathers(ci_b + 2, in_b, gsem_b)
        start_put(ci_b, out_b, osem_b)
        return 0

    lax.fori_loop(0, NCHUNK // 2, pair_body, 0)
    wait_put(NCHUNK - 2, out_a, osem_a)
    wait_put(NCHUNK - 1, out_b, osem_b)


_sc_embed = functools.partial(
    pl.kernel,
    mesh=plsc.VectorSubcoreMesh(core_axis_name="c", subcore_axis_name="s"),
    out_type=jax.ShapeDtypeStruct((B, L, H), jnp.float32),
    compiler_params=pltpu.CompilerParams(
        needs_layout_passes=False, use_tc_tiling_on_sc=False),
    scratch_types=[
        pltpu.VMEM((TOKH, CH), jnp.int32),
        pltpu.VMEM((CH, H), jnp.float32),
        pltpu.VMEM((CH, H), jnp.float32),
        pltpu.VMEM((RB, L, H), jnp.float32),
        pltpu.VMEM((RB, L, H), jnp.float32),
        pltpu.VMEM((L + 2, H), jnp.float32),
        pltpu.SemaphoreType.DMA,
        pltpu.SemaphoreType.DMA,
        pltpu.SemaphoreType.DMA,
        pltpu.SemaphoreType.DMA,
    ],
)(_sc_body)


def kernel(tokens, word_table, pos_table, ln_gamma, ln_beta):
    tok_w = tokens.reshape(NW, 2, TOKH, CH).astype(jnp.int32)
    posgb = jnp.concatenate(
        [pos_table, ln_gamma[None, :], ln_beta[None, :]], axis=0)
    return _sc_embed(tok_w, word_table, posgb)
